# baseline (speedup 1.0000x reference)
import functools

import jax
import jax.numpy as jnp
from jax import lax
from jax.experimental import pallas as pl
from jax.experimental.pallas import tpu as pltpu
from jax.experimental.pallas import tpu_sc as plsc

_BB = 64
_NW = 32
_SLAB_B = 2


def _proj_kernel(hidden_ref, wk_ref, bk_ref, wv_ref, bv_ref, wg_ref, bg_ref,
                 gb_ref, nkg_ref, nvg_ref, gate_ref):
    h = hidden_ref[...]
    dn = (((1,), (1,)), ((), ()))
    nk = jnp.tanh(lax.dot_general(h, wk_ref[...], dn,
                                  preferred_element_type=jnp.float32)
                  + bk_ref[...])
    nv = jnp.tanh(lax.dot_general(h, wv_ref[...], dn,
                                  preferred_element_type=jnp.float32)
                  + bv_ref[...])
    g = jax.nn.sigmoid(lax.dot_general(h, wg_ref[...], dn,
                                       preferred_element_type=jnp.float32)
                       + bg_ref[...])
    gb_ref[...] = g
    nkg_ref[...] = nk * g
    nvg_ref[...] = nv * g
    gate_ref[...] = g[:, :gate_ref.shape[1]]


def _keys_kernel(slot_ref, gb_ref, nkg_ref, keys_ref, out_keys_ref):
    slot = slot_ref[0]
    cur = keys_ref[:, slot, :]
    out_keys_ref[...] = keys_ref[...]
    out_keys_ref[:, slot, :] = cur * (1.0 - gb_ref[...]) + nkg_ref[...]


def _vals_kernel(vals_ref, gb_ref, nvg_ref, idx_ref,
                 out_ref,
                 slab0, slab1, curb, gbb, nvgb, idx_a, idx_b,
                 lsem0, lsem1, ssem0, ssem1, gsem):
    BS, D = vals_ref.shape
    S = 64
    nb = (BS // S) // _NW
    wid = lax.axis_index("s") * 2 + lax.axis_index("c")
    base_b = wid * nb
    rows_per_slab = _SLAB_B * S
    nslab = nb // _SLAB_B
    slab_rows0 = base_b * S

    def slab_copy(i, buf, lsem, ssem):
        r0 = slab_rows0 + i * 2 * rows_per_slab + (0 if buf is slab0 else rows_per_slab)
        @pl.when(i > 0)
        def _():
            pltpu.make_async_copy(buf, out_ref.at[pl.ds(r0, rows_per_slab)],
                                  ssem).wait()
        pltpu.async_copy(vals_ref.at[pl.ds(r0, rows_per_slab)], buf, lsem)

    def slab_store(i, buf, lsem, ssem):
        r0 = slab_rows0 + i * 2 * rows_per_slab + (0 if buf is slab0 else rows_per_slab)
        pltpu.make_async_copy(vals_ref.at[pl.ds(r0, rows_per_slab)], buf,
                              lsem).wait()
        pltpu.async_copy(buf, out_ref.at[pl.ds(r0, rows_per_slab)], ssem)

    def round_body(i, _):
        slab_copy(i, slab0, lsem0, ssem0)
        slab_copy(i, slab1, lsem1, ssem1)
        slab_store(i, slab0, lsem0, ssem0)
        slab_store(i, slab1, lsem1, ssem1)
        return _

    lax.fori_loop(0, nslab // 2, round_body, 0)
    pltpu.make_async_copy(slab0, out_ref.at[pl.ds(slab_rows0, rows_per_slab)],
                          ssem0).wait()
    pltpu.make_async_copy(slab1, out_ref.at[pl.ds(slab_rows0, rows_per_slab)],
                          ssem1).wait()

    half = nb // 2
    for h, idx_v in ((0, idx_a), (1, idx_b)):
        pltpu.sync_copy(idx_ref.at[wid * 2 + h], idx_v)
        pltpu.async_copy(vals_ref.at[idx_v], curb, gsem).wait()
        b0 = base_b + h * half
        pltpu.sync_copy(gb_ref.at[pl.ds(b0, half)], gbb)
        pltpu.sync_copy(nvg_ref.at[pl.ds(b0, half)], nvgb)

        def blend_row(b, _):
            for k in range(D // 16):
                sl = pl.ds(k * 16, 16)
                curb[b, sl] = (curb[b, sl] * (1.0 - gbb[b, sl])
                               + nvgb[b, sl])
            return _

        lax.fori_loop(0, half, blend_row, 0)
        pltpu.async_copy(curb, out_ref.at[idx_v], gsem).wait()


def kernel(hidden, working_keys, working_values, step, Wk, bk, Wv, bv, Wg, bg):
    B, S, D = working_keys.shape
    slot32 = jnp.asarray(step, jnp.int32) % S
    slot = slot32.reshape(1)

    smem = pl.BlockSpec(memory_space=pltpu.MemorySpace.SMEM)
    vmem = pl.BlockSpec(memory_space=pltpu.MemorySpace.VMEM)

    gb, nkg, nvg, gate = pl.pallas_call(
        _proj_kernel,
        out_shape=[
            jax.ShapeDtypeStruct((B, D), jnp.float32),
            jax.ShapeDtypeStruct((B, D), jnp.float32),
            jax.ShapeDtypeStruct((B, D), jnp.float32),
            jax.ShapeDtypeStruct((B, 128), jnp.float32),
        ],
        in_specs=[vmem] * 7,
        out_specs=[vmem] * 4,
    )(hidden, Wk, bk.reshape(1, D), Wv, bv.reshape(1, D),
      jnp.broadcast_to(Wg, (D, D)), jnp.broadcast_to(bg.reshape(1, 1), (1, D)))

    bblk = pl.BlockSpec((_BB, S, D), lambda i: (i, 0, 0))
    dblk = pl.BlockSpec((_BB, D), lambda i: (i, 0))
    out_keys = pl.pallas_call(
        _keys_kernel,
        grid=(B // _BB,),
        out_shape=jax.ShapeDtypeStruct((B, S, D), jnp.float32),
        in_specs=[smem, dblk, dblk, bblk],
        out_specs=bblk,
    )(slot, gb, nkg, working_keys)

    nb = B // _NW
    half = nb // 2
    idx = (jnp.arange(B, dtype=jnp.int32) * S + slot32).reshape(2 * _NW, half)
    v2 = working_values.reshape(B * S, D)
    rows_per_slab = _SLAB_B * S

    mesh = plsc.VectorSubcoreMesh(core_axis_name="c", subcore_axis_name="s")
    sc_call = pl.kernel(
        _vals_kernel,
        out_type=jax.ShapeDtypeStruct((B * S, D), jnp.float32),
        mesh=mesh,
        scratch_types=[
            pltpu.VMEM((rows_per_slab, D), jnp.float32),
            pltpu.VMEM((rows_per_slab, D), jnp.float32),
            pltpu.VMEM((half, D), jnp.float32),
            pltpu.VMEM((half, D), jnp.float32),
            pltpu.VMEM((half, D), jnp.float32),
            pltpu.VMEM((half,), jnp.int32),
            pltpu.VMEM((half,), jnp.int32),
            pltpu.SemaphoreType.DMA,
            pltpu.SemaphoreType.DMA,
            pltpu.SemaphoreType.DMA,
            pltpu.SemaphoreType.DMA,
            pltpu.SemaphoreType.DMA,
        ],
    )
    out_vals = sc_call(v2, gb, nvg, idx).reshape(B, S, D)

    return (out_keys, out_vals, gate[:, 0])


# speedup vs baseline: 1.0497x; 1.0497x over previous
import functools

import jax
import jax.numpy as jnp
from jax import lax
from jax.experimental import pallas as pl
from jax.experimental.pallas import tpu as pltpu
from jax.experimental.pallas import tpu_sc as plsc

_BB = 64
_NW = 32
_NBUF = 4
_SLAB_ROWS = 64


def _proj_kernel(hidden_ref, wk_ref, bk_ref, wv_ref, bv_ref, wg_ref, bg_ref,
                 gb_ref, nkg_ref, nvg_ref, gate_ref):
    h = hidden_ref[...]
    dn = (((1,), (1,)), ((), ()))
    nk = jnp.tanh(lax.dot_general(h, wk_ref[...], dn,
                                  preferred_element_type=jnp.float32)
                  + bk_ref[...])
    nv = jnp.tanh(lax.dot_general(h, wv_ref[...], dn,
                                  preferred_element_type=jnp.float32)
                  + bv_ref[...])
    g = jax.nn.sigmoid(lax.dot_general(h, wg_ref[...], dn,
                                       preferred_element_type=jnp.float32)
                       + bg_ref[...])
    gb_ref[...] = g
    nkg_ref[...] = nk * g
    nvg_ref[...] = nv * g
    gate_ref[...] = g[:, :gate_ref.shape[1]]


def _keys_kernel(slot_ref, gb_ref, nkg_ref, keys_ref, out_keys_ref):
    slot = slot_ref[0]
    cur = keys_ref[:, slot, :]
    out_keys_ref[...] = keys_ref[...]
    out_keys_ref[:, slot, :] = cur * (1.0 - gb_ref[...]) + nkg_ref[...]


def _vals_kernel(vals_ref, gb_ref, nvg_ref, idx_ref,
                 out_ref,
                 slabs, curb, gbb, nvgb, idx_v,
                 lsem0, lsem1, lsem2, lsem3, ssem0, ssem1, ssem2, ssem3,
                 gsem):
    BS, D = vals_ref.shape
    nb = (BS // 64) // _NW
    rows = nb * 64
    sid = lax.axis_index("s")
    wid = sid * 2 + lax.axis_index("c")
    row0 = wid * rows
    lsems = (lsem0, lsem1, lsem2, lsem3)
    ssems = (ssem0, ssem1, ssem2, ssem3)
    nround = rows // (_NBUF * _SLAB_ROWS)

    def src(i, b):
        return vals_ref.at[
            pl.ds(row0 + (i * _NBUF + b) * _SLAB_ROWS, _SLAB_ROWS)]

    def dst(i, b):
        return out_ref.at[
            pl.ds(row0 + (i * _NBUF + b) * _SLAB_ROWS, _SLAB_ROWS)]

    for b in range(_NBUF):
        pltpu.async_copy(src(0, b), slabs.at[sid, b], lsems[b])

    def round_body(i, c):
        for b in range(_NBUF):
            pltpu.make_async_copy(src(i, b), slabs.at[sid, b],
                                  lsems[b]).wait()
            pltpu.async_copy(slabs.at[sid, b], dst(i, b), ssems[b])

        @pl.when(i < nround - 1)
        def _():
            for b in range(_NBUF):
                pltpu.make_async_copy(slabs.at[sid, b], dst(i, b),
                                      ssems[b]).wait()
                pltpu.async_copy(src(i + 1, b), slabs.at[sid, b], lsems[b])
        return c

    lax.fori_loop(0, nround, round_body, 0)

    for b in range(_NBUF):
        pltpu.make_async_copy(slabs.at[sid, b], dst(0, b), ssems[b]).wait()

    half = nb // 2
    pltpu.sync_copy(idx_ref.at[wid], idx_v)
    for h in range(2):
        pltpu.async_copy(vals_ref.at[idx_v.at[h]], curb, gsem).wait()
        b0 = wid * nb + h * half
        pltpu.sync_copy(gb_ref.at[pl.ds(b0, half)], gbb)
        pltpu.sync_copy(nvg_ref.at[pl.ds(b0, half)], nvgb)

        def blend_row(bi, c):
            for k in range(D // 16):
                sl = pl.ds(k * 16, 16)
                curb[bi, sl] = (curb[bi, sl] * (1.0 - gbb[bi, sl])
                                + nvgb[bi, sl])
            return c

        lax.fori_loop(0, half, blend_row, 0)
        pltpu.async_copy(curb, out_ref.at[idx_v.at[h]], gsem).wait()


def kernel(hidden, working_keys, working_values, step, Wk, bk, Wv, bv, Wg, bg):
    B, S, D = working_keys.shape
    slot32 = jnp.asarray(step, jnp.int32) % S
    slot = slot32.reshape(1)

    smem = pl.BlockSpec(memory_space=pltpu.MemorySpace.SMEM)
    vmem = pl.BlockSpec(memory_space=pltpu.MemorySpace.VMEM)

    gb, nkg, nvg, gate = pl.pallas_call(
        _proj_kernel,
        out_shape=[
            jax.ShapeDtypeStruct((B, D), jnp.float32),
            jax.ShapeDtypeStruct((B, D), jnp.float32),
            jax.ShapeDtypeStruct((B, D), jnp.float32),
            jax.ShapeDtypeStruct((B, 128), jnp.float32),
        ],
        in_specs=[vmem] * 7,
        out_specs=[vmem] * 4,
    )(hidden, Wk, bk.reshape(1, D), Wv, bv.reshape(1, D),
      jnp.broadcast_to(Wg, (D, D)), jnp.broadcast_to(bg.reshape(1, 1), (1, D)))

    bblk = pl.BlockSpec((_BB, S, D), lambda i: (i, 0, 0))
    dblk = pl.BlockSpec((_BB, D), lambda i: (i, 0))
    out_keys = pl.pallas_call(
        _keys_kernel,
        grid=(B // _BB,),
        out_shape=jax.ShapeDtypeStruct((B, S, D), jnp.float32),
        in_specs=[smem, dblk, dblk, bblk],
        out_specs=bblk,
    )(slot, gb, nkg, working_keys)

    nb = B // _NW
    idx = (jnp.arange(B, dtype=jnp.int32) * S + slot32).reshape(_NW, 2,
                                                                nb // 2)
    v2 = working_values.reshape(B * S, D)

    mesh = plsc.VectorSubcoreMesh(core_axis_name="c", subcore_axis_name="s")
    sc_call = pl.kernel(
        _vals_kernel,
        out_type=jax.ShapeDtypeStruct((B * S, D), jnp.float32),
        mesh=mesh,
        scratch_types=[
            pltpu.VMEM_SHARED((_NW // 2, _NBUF, _SLAB_ROWS, D), jnp.float32),
            pltpu.VMEM((nb // 2, D), jnp.float32),
            pltpu.VMEM((nb // 2, D), jnp.float32),
            pltpu.VMEM((nb // 2, D), jnp.float32),
            pltpu.VMEM((2, nb // 2), jnp.int32),
        ] + [pltpu.SemaphoreType.DMA] * 9,
    )
    out_vals = sc_call(v2, gb, nvg, idx).reshape(B, S, D)

    return (out_keys, out_vals, gate[:, 0])


# speedup vs baseline: 1.0670x; 1.0165x over previous
import functools

import jax
import jax.numpy as jnp
from jax import lax
from jax.experimental import pallas as pl
from jax.experimental.pallas import tpu as pltpu
from jax.experimental.pallas import tpu_sc as plsc

_BB = 64
_NW = 32
_NBUF = 4
_SLAB_ROWS = 64


def _proj_kernel(hidden_ref, wk_ref, bk_ref, wv_ref, bv_ref, wg_ref, bg_ref,
                 gb_ref, nkg_ref, nvg_ref, gate_ref):
    h = hidden_ref[...]
    dn = (((1,), (1,)), ((), ()))
    nk = jnp.tanh(lax.dot_general(h, wk_ref[...], dn,
                                  preferred_element_type=jnp.float32)
                  + bk_ref[...])
    nv = jnp.tanh(lax.dot_general(h, wv_ref[...], dn,
                                  preferred_element_type=jnp.float32)
                  + bv_ref[...])
    g = jax.nn.sigmoid(lax.dot_general(h, wg_ref[...], dn,
                                       preferred_element_type=jnp.float32)
                       + bg_ref[...])
    gb_ref[...] = g
    nkg_ref[...] = nk * g
    nvg_ref[...] = nv * g
    gate_ref[...] = g[:, :gate_ref.shape[1]]


def _keys_kernel(slot_ref, gb_ref, nkg_ref, keys_ref, out_keys_ref):
    slot = slot_ref[0]
    cur = keys_ref[:, slot, :]
    out_keys_ref[...] = keys_ref[...]
    out_keys_ref[:, slot, :] = cur * (1.0 - gb_ref[...]) + nkg_ref[...]


def _vals_kernel(vals_ref, gb_ref, nvg_ref, idx_ref,
                 out_ref,
                 slabs, curb, hlfb, tmpb, idx_v,
                 lsem0, lsem1, lsem2, lsem3, ssem0, ssem1, ssem2, ssem3,
                 gsem, gsem2):
    BS, D = vals_ref.shape
    nb = (BS // 64) // _NW
    rows = nb * 64
    sid = lax.axis_index("s")
    wid = sid * 2 + lax.axis_index("c")
    row0 = wid * rows
    lsems = (lsem0, lsem1, lsem2, lsem3)
    ssems = (ssem0, ssem1, ssem2, ssem3)
    nround = rows // (_NBUF * _SLAB_ROWS)

    def src(i, b):
        return vals_ref.at[
            pl.ds(row0 + (i * _NBUF + b) * _SLAB_ROWS, _SLAB_ROWS)]

    def dst(i, b):
        return out_ref.at[
            pl.ds(row0 + (i * _NBUF + b) * _SLAB_ROWS, _SLAB_ROWS)]

    for b in range(_NBUF):
        pltpu.async_copy(src(0, b), slabs.at[sid, b], lsems[b])

    half = nb // 2
    pltpu.sync_copy(idx_ref.at[wid], idx_v)
    for h, acc in ((0, curb), (1, hlfb)):
        pltpu.async_copy(vals_ref.at[idx_v.at[h]], acc, gsem).wait()
        b0 = wid * nb + h * half

        def apply(op, c=None):
            def body(bi, cc):
                for k in range(D // 16):
                    sl = pl.ds(k * 16, 16)
                    acc[bi, sl] = op(acc[bi, sl], tmpb[bi, sl])
                return cc
            lax.fori_loop(0, half, body, 0)

        pltpu.sync_copy(gb_ref.at[pl.ds(b0, half)], tmpb)
        apply(lambda a, t: a * (1.0 - t))
        pltpu.sync_copy(nvg_ref.at[pl.ds(b0, half)], tmpb)
        apply(lambda a, t: a + t)

    def round_body(i, c):
        for b in range(_NBUF):
            pltpu.make_async_copy(src(i, b), slabs.at[sid, b],
                                  lsems[b]).wait()
            pltpu.async_copy(slabs.at[sid, b], dst(i, b), ssems[b])

        @pl.when(i < nround - 1)
        def _():
            for b in range(_NBUF):
                pltpu.make_async_copy(slabs.at[sid, b], dst(i, b),
                                      ssems[b]).wait()
                pltpu.async_copy(src(i + 1, b), slabs.at[sid, b], lsems[b])
        return c

    lax.fori_loop(0, nround, round_body, 0)

    for b in range(_NBUF):
        pltpu.make_async_copy(slabs.at[sid, b], dst(0, b), ssems[b]).wait()
    s0 = pltpu.async_copy(curb, out_ref.at[idx_v.at[0]], gsem)
    pltpu.async_copy(hlfb, out_ref.at[idx_v.at[1]], gsem2).wait()
    s0.wait()


def kernel(hidden, working_keys, working_values, step, Wk, bk, Wv, bv, Wg, bg):
    B, S, D = working_keys.shape
    slot32 = jnp.asarray(step, jnp.int32) % S
    slot = slot32.reshape(1)

    smem = pl.BlockSpec(memory_space=pltpu.MemorySpace.SMEM)
    vmem = pl.BlockSpec(memory_space=pltpu.MemorySpace.VMEM)

    gb, nkg, nvg, gate = pl.pallas_call(
        _proj_kernel,
        out_shape=[
            jax.ShapeDtypeStruct((B, D), jnp.float32),
            jax.ShapeDtypeStruct((B, D), jnp.float32),
            jax.ShapeDtypeStruct((B, D), jnp.float32),
            jax.ShapeDtypeStruct((B, 128), jnp.float32),
        ],
        in_specs=[vmem] * 7,
        out_specs=[vmem] * 4,
    )(hidden, Wk, bk.reshape(1, D), Wv, bv.reshape(1, D),
      jnp.broadcast_to(Wg, (D, D)), jnp.broadcast_to(bg.reshape(1, 1), (1, D)))

    bblk = pl.BlockSpec((_BB, S, D), lambda i: (i, 0, 0))
    dblk = pl.BlockSpec((_BB, D), lambda i: (i, 0))
    out_keys = pl.pallas_call(
        _keys_kernel,
        grid=(B // _BB,),
        out_shape=jax.ShapeDtypeStruct((B, S, D), jnp.float32),
        in_specs=[smem, dblk, dblk, bblk],
        out_specs=bblk,
    )(slot, gb, nkg, working_keys)

    nb = B // _NW
    idx = (jnp.arange(B, dtype=jnp.int32) * S + slot32).reshape(_NW, 2,
                                                                nb // 2)
    v2 = working_values.reshape(B * S, D)

    mesh = plsc.VectorSubcoreMesh(core_axis_name="c", subcore_axis_name="s")
    sc_call = pl.kernel(
        _vals_kernel,
        out_type=jax.ShapeDtypeStruct((B * S, D), jnp.float32),
        mesh=mesh,
        scratch_types=[
            pltpu.VMEM_SHARED((_NW // 2, _NBUF, _SLAB_ROWS, D), jnp.float32),
            pltpu.VMEM((nb // 2, D), jnp.float32),
            pltpu.VMEM((nb // 2, D), jnp.float32),
            pltpu.VMEM((nb // 2, D), jnp.float32),
            pltpu.VMEM((2, nb // 2), jnp.int32),
        ] + [pltpu.SemaphoreType.DMA] * 10,
    )
    out_vals = sc_call(v2, gb, nvg, idx).reshape(B, S, D)

    return (out_keys, out_vals, gate[:, 0])


# speedup vs baseline: 1.0672x; 1.0002x over previous
import functools

import jax
import jax.numpy as jnp
from jax import lax
from jax.experimental import pallas as pl
from jax.experimental.pallas import tpu as pltpu
from jax.experimental.pallas import tpu_sc as plsc

_BB = 64
_NW = 32
_NBUF = 4
_SLAB_ROWS = 64


def _proj_kernel(hidden_ref, wk_ref, bk_ref, wv_ref, bv_ref, wg_ref, bg_ref,
                 gb_ref, nkg_ref, nvg_ref, gate_ref):
    h = hidden_ref[...]
    dn = (((1,), (1,)), ((), ()))
    nk = jnp.tanh(lax.dot_general(h, wk_ref[...], dn,
                                  preferred_element_type=jnp.float32)
                  + bk_ref[...])
    nv = jnp.tanh(lax.dot_general(h, wv_ref[...], dn,
                                  preferred_element_type=jnp.float32)
                  + bv_ref[...])
    g = jax.nn.sigmoid(lax.dot_general(h, wg_ref[...], dn,
                                       preferred_element_type=jnp.float32)
                       + bg_ref[...])
    gb_ref[...] = g
    nkg_ref[...] = nk * g
    nvg_ref[...] = nv * g
    gate_ref[...] = g[:, :gate_ref.shape[1]]


def _keys_kernel(slot_ref, gb_ref, nkg_ref, keys_ref, out_keys_ref):
    slot = slot_ref[0]
    cur = keys_ref[:, slot, :]
    out_keys_ref[...] = keys_ref[...]
    out_keys_ref[:, slot, :] = cur * (1.0 - gb_ref[...]) + nkg_ref[...]


def _vals_kernel(vals_ref, gb_ref, nvg_ref, idx_ref,
                 out_ref,
                 slabs, curb, hlfb, tmpb, idx_v,
                 lsem0, lsem1, lsem2, lsem3, ssem0, ssem1, ssem2, ssem3,
                 gsem, gsem2):
    BS, D = vals_ref.shape
    nb = (BS // 64) // _NW
    rows = nb * 64
    sid = lax.axis_index("s")
    wid = sid * 2 + lax.axis_index("c")
    row0 = wid * rows
    lsems = (lsem0, lsem1, lsem2, lsem3)
    ssems = (ssem0, ssem1, ssem2, ssem3)
    nround = rows // (_NBUF * _SLAB_ROWS)

    def src(i, b):
        return vals_ref.at[
            pl.ds(row0 + (i * _NBUF + b) * _SLAB_ROWS, _SLAB_ROWS)]

    def dst(i, b):
        return out_ref.at[
            pl.ds(row0 + (i * _NBUF + b) * _SLAB_ROWS, _SLAB_ROWS)]

    for b in range(_NBUF):
        pltpu.async_copy(src(0, b), slabs.at[sid, b], lsems[b])

    half = nb // 2
    pltpu.sync_copy(idx_ref.at[wid], idx_v)
    for h, acc in ((0, curb), (1, hlfb)):
        pltpu.async_copy(vals_ref.at[idx_v.at[h]], acc, gsem).wait()
        b0 = wid * nb + h * half

        def apply(op, c=None):
            def body(bi, cc):
                for k in range(D // 16):
                    sl = pl.ds(k * 16, 16)
                    acc[bi, sl] = op(acc[bi, sl], tmpb[bi, sl])
                return cc
            lax.fori_loop(0, half, body, 0)

        pltpu.sync_copy(gb_ref.at[pl.ds(b0, half)], tmpb)
        apply(lambda a, t: a * (1.0 - t))
        pltpu.sync_copy(nvg_ref.at[pl.ds(b0, half)], tmpb)
        apply(lambda a, t: a + t)

    def fsrc(s):
        return vals_ref.at[pl.ds(row0 + s * _SLAB_ROWS, _SLAB_ROWS)]

    def fdst(s):
        return out_ref.at[pl.ds(row0 + s * _SLAB_ROWS, _SLAB_ROWS)]

    def round_body(i, c):
        for b in range(_NBUF):
            s = i * _NBUF + b
            b2 = (b + 2) % _NBUF

            @pl.when(i > 0)
            def _():
                pltpu.make_async_copy(slabs.at[sid, b], fdst(s - _NBUF),
                                      ssems[b]).wait()
                pltpu.async_copy(fsrc(s), slabs.at[sid, b], lsems[b])

            def step_store():
                pltpu.make_async_copy(fsrc(s - 2), slabs.at[sid, b2],
                                      lsems[b2]).wait()
                pltpu.async_copy(slabs.at[sid, b2], fdst(s - 2), ssems[b2])

            if b >= 2:
                step_store()
            else:
                pl.when(i > 0)(step_store)
        return c

    lax.fori_loop(0, nround, round_body, 0)

    last = nround * _NBUF
    for s in (last - 2, last - 1):
        b2 = s % _NBUF
        pltpu.make_async_copy(fsrc(s), slabs.at[sid, b2], lsems[b2]).wait()
        pltpu.async_copy(slabs.at[sid, b2], fdst(s), ssems[b2])
    for b in range(_NBUF):
        pltpu.make_async_copy(slabs.at[sid, b], fdst(0), ssems[b]).wait()
    s0 = pltpu.async_copy(curb, out_ref.at[idx_v.at[0]], gsem)
    pltpu.async_copy(hlfb, out_ref.at[idx_v.at[1]], gsem2).wait()
    s0.wait()


def kernel(hidden, working_keys, working_values, step, Wk, bk, Wv, bv, Wg, bg):
    B, S, D = working_keys.shape
    slot32 = jnp.asarray(step, jnp.int32) % S
    slot = slot32.reshape(1)

    smem = pl.BlockSpec(memory_space=pltpu.MemorySpace.SMEM)
    vmem = pl.BlockSpec(memory_space=pltpu.MemorySpace.VMEM)

    gb, nkg, nvg, gate = pl.pallas_call(
        _proj_kernel,
        out_shape=[
            jax.ShapeDtypeStruct((B, D), jnp.float32),
            jax.ShapeDtypeStruct((B, D), jnp.float32),
            jax.ShapeDtypeStruct((B, D), jnp.float32),
            jax.ShapeDtypeStruct((B, 128), jnp.float32),
        ],
        in_specs=[vmem] * 7,
        out_specs=[vmem] * 4,
    )(hidden, Wk, bk.reshape(1, D), Wv, bv.reshape(1, D),
      jnp.broadcast_to(Wg, (D, D)), jnp.broadcast_to(bg.reshape(1, 1), (1, D)))

    bblk = pl.BlockSpec((_BB, S, D), lambda i: (i, 0, 0))
    dblk = pl.BlockSpec((_BB, D), lambda i: (i, 0))
    out_keys = pl.pallas_call(
        _keys_kernel,
        grid=(B // _BB,),
        out_shape=jax.ShapeDtypeStruct((B, S, D), jnp.float32),
        in_specs=[smem, dblk, dblk, bblk],
        out_specs=bblk,
    )(slot, gb, nkg, working_keys)

    nb = B // _NW
    idx = (jnp.arange(B, dtype=jnp.int32) * S + slot32).reshape(_NW, 2,
                                                                nb // 2)
    v2 = working_values.reshape(B * S, D)

    mesh = plsc.VectorSubcoreMesh(core_axis_name="c", subcore_axis_name="s")
    sc_call = pl.kernel(
        _vals_kernel,
        out_type=jax.ShapeDtypeStruct((B * S, D), jnp.float32),
        mesh=mesh,
        scratch_types=[
            pltpu.VMEM_SHARED((_NW // 2, _NBUF, _SLAB_ROWS, D), jnp.float32),
            pltpu.VMEM((nb // 2, D), jnp.float32),
            pltpu.VMEM((nb // 2, D), jnp.float32),
            pltpu.VMEM((nb // 2, D), jnp.float32),
            pltpu.VMEM((2, nb // 2), jnp.int32),
        ] + [pltpu.SemaphoreType.DMA] * 10,
    )
    out_vals = sc_call(v2, gb, nvg, idx).reshape(B, S, D)

    return (out_keys, out_vals, gate[:, 0])


# speedup vs baseline: 1.1662x; 1.0928x over previous
import jax
import jax.numpy as jnp
from jax import lax
from jax.experimental import pallas as pl
from jax.experimental.pallas import tpu as pltpu

_BB = 64


def _writer_kernel(slot_ref, hidden_ref, wk_ref, bk_ref, wv_ref, bv_ref,
                   wg_ref, bg_ref, keys_ref, vals_ref,
                   out_keys_ref, out_vals_ref, gate_ref):
    S = keys_ref.shape[1]
    slot = slot_ref[0]

    h = hidden_ref[...]
    dn = (((1,), (1,)), ((), ()))
    nk = jnp.tanh(lax.dot_general(h, wk_ref[...], dn,
                                  preferred_element_type=jnp.float32)
                  + bk_ref[...])
    nv = jnp.tanh(lax.dot_general(h, wv_ref[...], dn,
                                  preferred_element_type=jnp.float32)
                  + bv_ref[...])
    g = jax.nn.sigmoid(lax.dot_general(h, wg_ref[...], dn,
                                       preferred_element_type=jnp.float32)
                       + bg_ref[...])

    cur_k = keys_ref[:, slot, :]
    cur_v = vals_ref[:, slot, :]
    blend_k = cur_k * (1.0 - g) + nk * g
    blend_v = cur_v * (1.0 - g) + nv * g

    out_keys_ref[...] = keys_ref[...]
    out_vals_ref[...] = vals_ref[...]
    out_keys_ref[:, slot, :] = blend_k
    out_vals_ref[:, slot, :] = blend_v
    gate_ref[...] = g[:, :gate_ref.shape[1]]


def kernel(hidden, working_keys, working_values, step, Wk, bk, Wv, bv, Wg, bg):
    B, S, D = working_keys.shape
    slot = (jnp.asarray(step, jnp.int32) % S).reshape(1)

    smem = pl.BlockSpec(memory_space=pltpu.MemorySpace.SMEM)
    full = lambda shape: pl.BlockSpec(shape, lambda i: (0,) * len(shape))
    bblk = pl.BlockSpec((_BB, S, D), lambda i: (i, 0, 0))

    out_keys, out_vals, gate = pl.pallas_call(
        _writer_kernel,
        grid=(B // _BB,),
        out_shape=[
            jax.ShapeDtypeStruct((B, S, D), jnp.float32),
            jax.ShapeDtypeStruct((B, S, D), jnp.float32),
            jax.ShapeDtypeStruct((B, 128), jnp.float32),
        ],
        in_specs=[
            smem,
            pl.BlockSpec((_BB, D), lambda i: (i, 0)),
            full((D, D)), full((1, D)), full((D, D)), full((1, D)),
            full((D, D)), full((1, D)),
            bblk, bblk,
        ],
        out_specs=[bblk, bblk, pl.BlockSpec((_BB, 128), lambda i: (i, 0))],
    )(slot, hidden, Wk, bk.reshape(1, D), Wv, bv.reshape(1, D),
      jnp.broadcast_to(Wg, (D, D)), jnp.broadcast_to(bg.reshape(1, 1), (1, D)),
      working_keys, working_values)

    return (out_keys, out_vals, gate[:, 0])
